# baseline (speedup 1.0000x reference)
import jax
import jax.numpy as jnp
from jax.experimental import pallas as pl
from jax.experimental.pallas import tpu as pltpu

_NUM_HEADS = 12
_HEAD_DIM = 64
_POLY_DIM = 256
_MEM_HID = 512
_HIDDEN = 768


def _atlas_body(x_ref, wqt_ref, w1t_ref, b1_ref, w2t_ref, b2_ref, o_ref):
    x = x_ref[...]
    t = x.shape[0]
    qt = jax.lax.dot_general(wqt_ref[...], x, (((1,), (1,)), ((), ())),
                             preferred_element_type=jnp.float32)
    f1 = jnp.clip(qt, -10.0, 10.0)
    f2 = f1 * f1
    f3 = f2 * f1

    w1t = w1t_ref[...]
    b1 = pltpu.repeat(b1_ref[...], t // 128, axis=1)
    w2t = w2t_ref[...]
    b2 = b2_ref[...]

    outs_t = []
    for j in range(_NUM_HEADS):
        rows = slice(j * _HEAD_DIM, (j + 1) * _HEAD_DIM)
        feats_t = jnp.concatenate([f1[rows, :], f2[rows, :], f3[rows, :]],
                                  axis=0)
        ht = jnp.dot(w1t, feats_t, preferred_element_type=jnp.float32) + b1
        ht = jnp.maximum(ht, 0.0)
        outs_t.append(
            jnp.dot(w2t, ht, preferred_element_type=jnp.float32))
    o_t = jnp.concatenate(outs_t, axis=0)
    o_ref[...] = o_t.T + b2


def kernel(hidden_states, Wq, coeffs, W1, b1, W2, b2):
    B, S, H = hidden_states.shape
    x = hidden_states.reshape(B * S, H)
    c0, c1, c2, c3 = coeffs[0], coeffs[1], coeffs[2], coeffs[3]
    scale = jnp.concatenate([
        jnp.full((_HEAD_DIM,), c1), jnp.full((_HEAD_DIM,), c2),
        jnp.full((_HEAD_DIM,), c3)])
    w1t = (W1[_HEAD_DIM:, :] * scale[:, None]).T
    b1_eff = b1 + c0 * jnp.sum(W1[:_HEAD_DIM, :], axis=0)
    b1bc = jnp.tile(b1_eff[:, None], (1, 128))
    w2t = W2[:, :_HEAD_DIM].T
    b2r = jnp.tile(b2[:_HEAD_DIM].reshape(1, _HEAD_DIM), (1, _NUM_HEADS))

    T = 1024
    grid = (B * S // T,)
    out = pl.pallas_call(
        _atlas_body,
        grid=grid,
        in_specs=[
            pl.BlockSpec((T, H), lambda i: (i, 0)),
            pl.BlockSpec((H, H), lambda i: (0, 0)),
            pl.BlockSpec((_MEM_HID, _POLY_DIM - _HEAD_DIM),
                         lambda i: (0, 0)),
            pl.BlockSpec((_MEM_HID, 128), lambda i: (0, 0)),
            pl.BlockSpec((_HEAD_DIM, _MEM_HID), lambda i: (0, 0)),
            pl.BlockSpec((1, _HIDDEN), lambda i: (0, 0)),
        ],
        out_specs=pl.BlockSpec((T, H), lambda i: (i, 0)),
        out_shape=jax.ShapeDtypeStruct((B * S, H), jnp.float32),
        compiler_params=pltpu.CompilerParams(
            dimension_semantics=("parallel",),
        ),
        name="atlas_attention_fused",
    )(x, Wq.T, w1t, b1bc, w2t, b2r)
    return out.reshape(B, S, _NUM_HEADS * _HEAD_DIM)


# speedup vs baseline: 1.0275x; 1.0275x over previous
import jax
import jax.numpy as jnp
from jax.experimental import pallas as pl
from jax.experimental.pallas import tpu as pltpu

_NUM_HEADS = 12
_HEAD_DIM = 64
_POLY_DIM = 256
_MEM_HID = 512
_HIDDEN = 768


def _atlas_body(x_ref, wq_ref, w1_ref, b1_ref, w2_ref, b2_ref, o_ref):
    x = x_ref[...]
    q = jnp.dot(x, wq_ref[...], preferred_element_type=jnp.float32)
    f1 = jnp.clip(q, -10.0, 10.0)
    f2 = f1 * f1
    f3 = f2 * f1

    w1p = w1_ref[...]
    b1 = b1_ref[...]
    w2 = w2_ref[...]
    b2 = b2_ref[...]

    outs = []
    for j in range(_NUM_HEADS):
        sl = slice(j * _HEAD_DIM, (j + 1) * _HEAD_DIM)
        feats = jnp.concatenate([f1[:, sl], f2[:, sl], f3[:, sl]],
                                axis=-1).astype(jnp.bfloat16)
        h = jnp.dot(feats, w1p, preferred_element_type=jnp.float32) + b1
        h = jnp.maximum(h, 0.0).astype(jnp.bfloat16)
        outs.append(jnp.dot(h, w2, preferred_element_type=jnp.float32) + b2)
    o_ref[...] = jnp.concatenate(outs, axis=-1)


def kernel(hidden_states, Wq, coeffs, W1, b1, W2, b2):
    B, S, H = hidden_states.shape
    x = hidden_states.reshape(B * S, H)
    rowscale = jnp.repeat(coeffs[1:], _HEAD_DIM)[:, None]
    w1p = (W1[_HEAD_DIM:, :] * rowscale).astype(jnp.bfloat16)
    b1_eff = b1 + coeffs[0] * jnp.sum(W1[:_HEAD_DIM, :], axis=0)
    w2s = W2[:, :_HEAD_DIM].astype(jnp.bfloat16)
    b1r = b1_eff.reshape(1, _MEM_HID)
    b2r = b2[:_HEAD_DIM].reshape(1, _HEAD_DIM)

    T = 1024
    grid = (B * S // T,)
    out = pl.pallas_call(
        _atlas_body,
        grid=grid,
        in_specs=[
            pl.BlockSpec((T, H), lambda i: (i, 0)),
            pl.BlockSpec((H, H), lambda i: (0, 0)),
            pl.BlockSpec((_POLY_DIM - _HEAD_DIM, _MEM_HID),
                         lambda i: (0, 0)),
            pl.BlockSpec((1, _MEM_HID), lambda i: (0, 0)),
            pl.BlockSpec((_MEM_HID, _HEAD_DIM), lambda i: (0, 0)),
            pl.BlockSpec((1, _HEAD_DIM), lambda i: (0, 0)),
        ],
        out_specs=pl.BlockSpec((T, H), lambda i: (i, 0)),
        out_shape=jax.ShapeDtypeStruct((B * S, H), jnp.float32),
        compiler_params=pltpu.CompilerParams(
            dimension_semantics=("parallel",),
        ),
        name="atlas_attention_fused",
    )(x, Wq, w1p, b1r, w2s, b2r)
    return out.reshape(B, S, _NUM_HEADS * _HEAD_DIM)


# speedup vs baseline: 1.0784x; 1.0496x over previous
import jax
import jax.numpy as jnp
from jax.experimental import pallas as pl
from jax.experimental.pallas import tpu as pltpu

_NUM_HEADS = 12
_HEAD_DIM = 64
_POLY_DIM = 256
_MEM_HID = 512
_HIDDEN = 768


def _atlas_body(coeffs_ref, x_ref, wq_ref, w1_ref, b1_ref, w2_ref, b2_ref,
                o_ref):
    c0 = coeffs_ref[0]

    x = x_ref[...]
    q = jnp.dot(x, wq_ref[...], preferred_element_type=jnp.float32)
    f1 = jnp.clip(q, -10.0, 10.0)
    f2 = f1 * f1
    f3 = f2 * f1

    w1c = w1_ref[...]
    w1p = w1c[_HEAD_DIM:, :]
    b1 = b1_ref[...] + c0 * jnp.sum(w1c[:_HEAD_DIM, :], axis=0, keepdims=True)
    w2 = w2_ref[...]
    b2 = b2_ref[...]

    outs = []
    for j in range(_NUM_HEADS):
        sl = slice(j * _HEAD_DIM, (j + 1) * _HEAD_DIM)
        feats = jnp.concatenate([f1[:, sl], f2[:, sl], f3[:, sl]], axis=-1)
        h = jnp.dot(feats, w1p, preferred_element_type=jnp.float32) + b1
        h = jnp.maximum(h, 0.0)
        outs.append(jnp.dot(h, w2, preferred_element_type=jnp.float32) + b2)
    o_ref[...] = jnp.concatenate(outs, axis=-1)


def kernel(hidden_states, Wq, coeffs, W1, b1, W2, b2):
    B, S, H = hidden_states.shape
    x = hidden_states.reshape(B * S, H)
    rowscale = jnp.repeat(coeffs, _HEAD_DIM)[:, None]
    w1c = W1 * jnp.where(jnp.arange(_POLY_DIM)[:, None] < _HEAD_DIM,
                         1.0, rowscale)
    w2s = W2[:, :_HEAD_DIM]
    b1r = b1.reshape(1, _MEM_HID)
    b2r = b2[:_HEAD_DIM].reshape(1, _HEAD_DIM)

    T = 1024
    grid = (B * S // T,)
    out = pl.pallas_call(
        _atlas_body,
        grid=grid,
        in_specs=[
            pl.BlockSpec(memory_space=pltpu.SMEM),
            pl.BlockSpec((T, H), lambda i: (i, 0)),
            pl.BlockSpec((H, H), lambda i: (0, 0)),
            pl.BlockSpec((_POLY_DIM, _MEM_HID), lambda i: (0, 0)),
            pl.BlockSpec((1, _MEM_HID), lambda i: (0, 0)),
            pl.BlockSpec((_MEM_HID, _HEAD_DIM), lambda i: (0, 0)),
            pl.BlockSpec((1, _HEAD_DIM), lambda i: (0, 0)),
        ],
        out_specs=pl.BlockSpec((T, H), lambda i: (i, 0)),
        out_shape=jax.ShapeDtypeStruct((B * S, H), jnp.float32),
        compiler_params=pltpu.CompilerParams(
            dimension_semantics=("parallel",),
        ),
        name="atlas_attention_fused",
    )(coeffs, x, Wq, w1c, b1r, w2s, b2r)
    return out.reshape(B, S, _NUM_HEADS * _HEAD_DIM)
